# Initial kernel scaffold; baseline (speedup 1.0000x reference)
import jax
import jax.numpy as jnp
from jax.experimental import pallas as pl


def kernel(text_table, W1, b1, W2, b2, codec_table, text_ids, codec_ids):
    raise NotImplementedError("write your pallas kernel here")


# speedup vs baseline: 2.5009x; 2.5009x over previous
import functools

import jax
import jax.numpy as jnp
from jax import lax
from jax.experimental import pallas as pl
from jax.experimental.pallas import tpu as pltpu
from jax.experimental.pallas import tpu_sc as plsc

TEXT_HIDDEN = 1024
HIDDEN = 1024

NC = 2
NS = 16
NW = NC * NS

CHUNK = 64


def _sc_gather_body(ntext_chunks, ncodec_chunks,
                    text_table, codec_table, text_idx, codec_idx,
                    text_out, codec_out,
                    idx_a, idx_b, rows, sem):
    wid = lax.axis_index("s") * NC + lax.axis_index("c")

    def do_chunk(table, idx_hbm, out_hbm, nchunks, c, idx_v):
        base = wid * (nchunks * CHUNK) + c * CHUNK
        pltpu.sync_copy(idx_hbm.at[pl.ds(base, CHUNK)], idx_v)
        pltpu.async_copy(table.at[idx_v], rows, sem).wait()
        pltpu.sync_copy(rows, out_hbm.at[pl.ds(base, CHUNK)])

    for c in range(ntext_chunks):
        do_chunk(text_table, text_idx, text_out, ntext_chunks, c, idx_a)
    for c in range(ncodec_chunks):
        do_chunk(codec_table, codec_idx, codec_out, ncodec_chunks, c, idx_b)


def _sc_gather(text_table, codec_table, text_idx, codec_idx):
    n = text_idx.shape[0]
    assert n % (NW * CHUNK) == 0
    nchunks = n // (NW * CHUNK)
    mesh = plsc.VectorSubcoreMesh(core_axis_name="c", subcore_axis_name="s")
    fn = pl.kernel(
        functools.partial(_sc_gather_body, nchunks, nchunks),
        out_type=(
            jax.ShapeDtypeStruct((n, TEXT_HIDDEN), jnp.float32),
            jax.ShapeDtypeStruct((n, HIDDEN), jnp.float32),
        ),
        mesh=mesh,
        scratch_types=[
            pltpu.VMEM((CHUNK,), jnp.int32),
            pltpu.VMEM((CHUNK,), jnp.int32),
            pltpu.VMEM((CHUNK, TEXT_HIDDEN), jnp.float32),
            pltpu.SemaphoreType.DMA,
        ],
    )
    return fn(text_table, codec_table, text_idx, codec_idx)


def _mlp_block(x_ref, w1_ref, b1_ref, w2_ref, b2_ref, o_ref):
    x = x_ref[...]
    h = jnp.dot(x, w1_ref[...], preferred_element_type=jnp.float32)
    h = h + b1_ref[...]
    h = h * jax.nn.sigmoid(h)
    o = jnp.dot(h, w2_ref[...], preferred_element_type=jnp.float32)
    o_ref[...] = o + b2_ref[...]


def _tc_mlp(x, W1, b1, W2, b2):
    n = x.shape[0]
    blk = 512
    grid = (n // blk,)
    return pl.pallas_call(
        _mlp_block,
        grid=grid,
        in_specs=[
            pl.BlockSpec((blk, TEXT_HIDDEN), lambda i: (i, 0)),
            pl.BlockSpec((TEXT_HIDDEN, TEXT_HIDDEN), lambda i: (0, 0)),
            pl.BlockSpec((1, TEXT_HIDDEN), lambda i: (0, 0)),
            pl.BlockSpec((TEXT_HIDDEN, HIDDEN), lambda i: (0, 0)),
            pl.BlockSpec((1, HIDDEN), lambda i: (0, 0)),
        ],
        out_specs=pl.BlockSpec((blk, HIDDEN), lambda i: (i, 0)),
        out_shape=jax.ShapeDtypeStruct((n, HIDDEN), jnp.float32),
    )(x, W1, b1.reshape(1, -1), W2, b2.reshape(1, -1))


def kernel(text_table, W1, b1, W2, b2, codec_table, text_ids, codec_ids):
    B, T = text_ids.shape
    n = B * T
    text_idx = text_ids.reshape(n).astype(jnp.int32)
    codec_idx = codec_ids.reshape(n).astype(jnp.int32)
    text_embeds, codec_embeds = _sc_gather(
        text_table, codec_table, text_idx, codec_idx)
    text_out = _tc_mlp(text_embeds, W1, b1, W2, b2)
    return (text_out.reshape(B, T, HIDDEN),
            codec_embeds.reshape(B, codec_ids.shape[1], HIDDEN))


# speedup vs baseline: 2.9340x; 1.1732x over previous
import functools

import jax
import jax.numpy as jnp
from jax import lax
from jax.experimental import pallas as pl
from jax.experimental.pallas import tpu as pltpu
from jax.experimental.pallas import tpu_sc as plsc

TEXT_HIDDEN = 1024
HIDDEN = 1024

NC = 2
NS = 16
NW = NC * NS

CHUNK = 64


def _sc_gather_body(nchunks, table, idx_hbm, out_hbm, idx_v, rows, sem):
    wid = lax.axis_index("s") * NC + lax.axis_index("c")
    for c in range(nchunks):
        base = wid * (nchunks * CHUNK) + c * CHUNK
        pltpu.sync_copy(idx_hbm.at[pl.ds(base, CHUNK)], idx_v)
        pltpu.async_copy(table.at[idx_v], rows, sem).wait()
        pltpu.sync_copy(rows, out_hbm.at[pl.ds(base, CHUNK)])


def _sc_gather(table, idx):
    n = idx.shape[0]
    d = table.shape[1]
    assert n % (NW * CHUNK) == 0
    nchunks = n // (NW * CHUNK)
    mesh = plsc.VectorSubcoreMesh(core_axis_name="c", subcore_axis_name="s")
    fn = pl.kernel(
        functools.partial(_sc_gather_body, nchunks),
        out_type=jax.ShapeDtypeStruct((n, d), jnp.float32),
        mesh=mesh,
        scratch_types=[
            pltpu.VMEM((CHUNK,), jnp.int32),
            pltpu.VMEM((CHUNK, d), jnp.float32),
            pltpu.SemaphoreType.DMA,
        ],
    )
    return fn(table, idx)


def _mlp_block(x_ref, w1_ref, b1_ref, w2_ref, b2_ref, o_ref):
    x = x_ref[...]
    h = jnp.dot(x, w1_ref[...], preferred_element_type=jnp.float32)
    h = h + b1_ref[...]
    h = h * jax.nn.sigmoid(h)
    o = jnp.dot(h, w2_ref[...], preferred_element_type=jnp.float32)
    o_ref[...] = o + b2_ref[...]


def _tc_mlp(x, W1, b1, W2, b2):
    n = x.shape[0]
    blk = 512
    grid = (n // blk,)
    return pl.pallas_call(
        _mlp_block,
        grid=grid,
        in_specs=[
            pl.BlockSpec((blk, TEXT_HIDDEN), lambda i: (i, 0)),
            pl.BlockSpec((TEXT_HIDDEN, TEXT_HIDDEN), lambda i: (0, 0)),
            pl.BlockSpec((1, TEXT_HIDDEN), lambda i: (0, 0)),
            pl.BlockSpec((TEXT_HIDDEN, HIDDEN), lambda i: (0, 0)),
            pl.BlockSpec((1, HIDDEN), lambda i: (0, 0)),
        ],
        out_specs=pl.BlockSpec((blk, HIDDEN), lambda i: (i, 0)),
        out_shape=jax.ShapeDtypeStruct((n, HIDDEN), jnp.float32),
    )(x, W1, b1.reshape(1, -1), W2, b2.reshape(1, -1))


def kernel(text_table, W1, b1, W2, b2, codec_table, text_ids, codec_ids):
    B, T = text_ids.shape
    n = B * T
    text_idx = text_ids.reshape(n).astype(jnp.int32)
    codec_idx = codec_ids.reshape(n).astype(jnp.int32)
    text_embeds = _sc_gather(text_table, text_idx)
    codec_embeds = _sc_gather(codec_table, codec_idx)
    text_out = _tc_mlp(text_embeds, W1, b1, W2, b2)
    return (text_out.reshape(B, T, HIDDEN),
            codec_embeds.reshape(B, codec_ids.shape[1], HIDDEN))


# speedup vs baseline: 2.9842x; 1.0171x over previous
import functools

import jax
import jax.numpy as jnp
from jax import lax
from jax.experimental import pallas as pl
from jax.experimental.pallas import tpu as pltpu
from jax.experimental.pallas import tpu_sc as plsc

TEXT_HIDDEN = 1024
HIDDEN = 1024

NC = 2
NS = 16
NW = NC * NS

CHUNK = 32
NBUF = 3


def _sc_gather_body(nchunks, table, idx_hbm, out_hbm, idx_v, rows, *sems):
    gsems, wsems = sems[:NBUF], sems[NBUF:]
    wid = lax.axis_index("s") * NC + lax.axis_index("c")
    pltpu.sync_copy(idx_hbm.at[wid], idx_v)
    g = [None] * nchunks
    w = [None] * nchunks

    def start_gather(c):
        g[c] = pltpu.async_copy(
            table.at[idx_v.at[c]], rows.at[c % NBUF], gsems[c % NBUF])

    def start_write(c):
        base = wid * (nchunks * CHUNK) + c * CHUNK
        w[c] = pltpu.async_copy(
            rows.at[c % NBUF], out_hbm.at[pl.ds(base, CHUNK)], wsems[c % NBUF])

    for c in range(min(NBUF, nchunks)):
        start_gather(c)
    for c in range(nchunks):
        g[c].wait()
        start_write(c)
        if c + NBUF < nchunks:
            w[c].wait()
            start_gather(c + NBUF)
    for c in range(max(0, nchunks - NBUF), nchunks):
        w[c].wait()


def _sc_gather(table, idx):
    n = idx.shape[0]
    d = table.shape[1]
    assert n % (NW * CHUNK) == 0
    nchunks = n // (NW * CHUNK)
    mesh = plsc.VectorSubcoreMesh(core_axis_name="c", subcore_axis_name="s")
    fn = pl.kernel(
        functools.partial(_sc_gather_body, nchunks),
        out_type=jax.ShapeDtypeStruct((n, d), jnp.float32),
        mesh=mesh,
        scratch_types=[
            pltpu.VMEM((nchunks, CHUNK), jnp.int32),
            pltpu.VMEM((NBUF, CHUNK, d), jnp.float32),
        ] + [pltpu.SemaphoreType.DMA] * (2 * NBUF),
    )
    return fn(table, idx.reshape(NW, nchunks, CHUNK))


def _mlp_block(x_ref, w1_ref, b1_ref, w2_ref, b2_ref, o_ref):
    x = x_ref[...]
    h = jnp.dot(x, w1_ref[...], preferred_element_type=jnp.float32)
    h = h + b1_ref[...]
    h = h * jax.nn.sigmoid(h)
    o = jnp.dot(h, w2_ref[...], preferred_element_type=jnp.float32)
    o_ref[...] = o + b2_ref[...]


def _tc_mlp(x, W1, b1, W2, b2):
    n = x.shape[0]
    blk = 512
    grid = (n // blk,)
    return pl.pallas_call(
        _mlp_block,
        grid=grid,
        in_specs=[
            pl.BlockSpec((blk, TEXT_HIDDEN), lambda i: (i, 0)),
            pl.BlockSpec((TEXT_HIDDEN, TEXT_HIDDEN), lambda i: (0, 0)),
            pl.BlockSpec((1, TEXT_HIDDEN), lambda i: (0, 0)),
            pl.BlockSpec((TEXT_HIDDEN, HIDDEN), lambda i: (0, 0)),
            pl.BlockSpec((1, HIDDEN), lambda i: (0, 0)),
        ],
        out_specs=pl.BlockSpec((blk, HIDDEN), lambda i: (i, 0)),
        out_shape=jax.ShapeDtypeStruct((n, HIDDEN), jnp.float32),
    )(x, W1, b1.reshape(1, -1), W2, b2.reshape(1, -1))


def kernel(text_table, W1, b1, W2, b2, codec_table, text_ids, codec_ids):
    B, T = text_ids.shape
    n = B * T
    text_idx = text_ids.reshape(n).astype(jnp.int32)
    codec_idx = codec_ids.reshape(n).astype(jnp.int32)
    text_embeds = _sc_gather(text_table, text_idx)
    codec_embeds = _sc_gather(codec_table, codec_idx)
    text_out = _tc_mlp(text_embeds, W1, b1, W2, b2)
    return (text_out.reshape(B, T, HIDDEN),
            codec_embeds.reshape(B, codec_ids.shape[1], HIDDEN))
